# Initial kernel scaffold; baseline (speedup 1.0000x reference)
import jax
import jax.numpy as jnp
from jax.experimental import pallas as pl


def kernel(x_lane_segment, x_lane, x_movement, x_phase, ea_ls_lane, ea_lane_down, ea_lane_up, ea_mm, ea_mp, ea_pp, ei_ls_lane, ei_lane_down, ei_lane_up, ei_mm, ei_mp, ei_pp, ei_mov_int, ei_phase_int, lane_Wq, lane_Wk, lane_Wv, lane_Wskip, lane_bskip, mvd_Wq, mvd_Wk, mvd_Wv, mvd_Wskip, mvd_bskip, mvu_Wq, mvu_Wk, mvu_Wv, mvu_Wskip, mvu_bskip, mm_Wq, mm_Wk, mm_Wv, mm_Wskip, mm_bskip, mp_Wq, mp_Wk, mp_Wv, mp_Wskip, mp_bskip, pp_Wq, pp_Wk, pp_Wv, pp_Wskip, pp_bskip, lane_resW, lane_resb, mov_resW, mov_resb, ph_resW, ph_resb, W_sv, b_sv):
    raise NotImplementedError("write your pallas kernel here")


# speedup vs baseline: 12.0073x; 12.0073x over previous
import functools
import jax
import jax.numpy as jnp
from jax.experimental import pallas as pl

HID = 64
HEADS = 8
DH = 8
N_INT = 10000


def _res_block(h, Ws, bs):
    nres = Ws.shape[0]

    def body(h_ref, W_ref, b_ref, o_ref):
        hh = h_ref[...]
        for i in range(nres):
            hh = hh + jnp.maximum(hh @ W_ref[i] + b_ref[i], 0.0)
        o_ref[...] = hh

    n = h.shape[0]
    BN = 512
    return pl.pallas_call(
        body,
        grid=(pl.cdiv(n, BN),),
        in_specs=[
            pl.BlockSpec((BN, HID), lambda i: (i, 0)),
            pl.BlockSpec((nres, HID, HID), lambda i: (0, 0, 0)),
            pl.BlockSpec((nres, HID), lambda i: (0, 0)),
        ],
        out_specs=pl.BlockSpec((BN, HID), lambda i: (i, 0)),
        out_shape=jax.ShapeDtypeStruct((n, HID), jnp.float32),
    )(h, Ws, bs)


def _tconv(x_src, x_dst, eattr, ei, Wq, Wk, Wv, Wskip, bskip):
    src, dst = ei[0], ei[1]
    n_dst = x_dst.shape[0]
    d_src = x_src.shape[1]
    q = x_dst @ (Wq * (1.0 / jnp.sqrt(float(DH))))
    Ks = x_src @ Wk[:d_src]
    Vs = x_src @ Wv[:d_src]
    k = Ks[src] + eattr @ Wk[d_src:]
    v = Vs[src] + eattr @ Wv[d_src:]
    z = (q[dst] * k).reshape(-1, HEADS, DH).sum(-1)
    w = jnp.exp(z)
    den = jax.ops.segment_sum(w, dst, num_segments=n_dst) + 1e-16
    num = jax.ops.segment_sum(jnp.repeat(w, DH, axis=1) * v, dst,
                              num_segments=n_dst)
    return num / jnp.repeat(den, DH, axis=1) + x_dst @ Wskip + bskip


def kernel(x_lane_segment, x_lane, x_movement, x_phase,
           ea_ls_lane, ea_lane_down, ea_lane_up, ea_mm, ea_mp, ea_pp,
           ei_ls_lane, ei_lane_down, ei_lane_up, ei_mm, ei_mp, ei_pp,
           ei_mov_int, ei_phase_int,
           lane_Wq, lane_Wk, lane_Wv, lane_Wskip, lane_bskip,
           mvd_Wq, mvd_Wk, mvd_Wv, mvd_Wskip, mvd_bskip,
           mvu_Wq, mvu_Wk, mvu_Wv, mvu_Wskip, mvu_bskip,
           mm_Wq, mm_Wk, mm_Wv, mm_Wskip, mm_bskip,
           mp_Wq, mp_Wk, mp_Wv, mp_Wskip, mp_bskip,
           pp_Wq, pp_Wk, pp_Wv, pp_Wskip, pp_bskip,
           lane_resW, lane_resb, mov_resW, mov_resb, ph_resW, ph_resb,
           W_sv, b_sv):
    h_lane = _tconv(x_lane_segment, x_lane, ea_ls_lane, ei_ls_lane,
                    lane_Wq, lane_Wk, lane_Wv, lane_Wskip, lane_bskip)
    h_lane = _res_block(h_lane, lane_resW, lane_resb)
    h_mov = _tconv(h_lane, x_movement, ea_lane_down, ei_lane_down,
                   mvd_Wq, mvd_Wk, mvd_Wv, mvd_Wskip, mvd_bskip)
    h_mov = h_mov + _tconv(h_lane, x_movement, ea_lane_up, ei_lane_up,
                           mvu_Wq, mvu_Wk, mvu_Wv, mvu_Wskip, mvu_bskip)
    for hop in range(2):
        h_mov = _tconv(h_mov, h_mov, ea_mm, ei_mm, mm_Wq[hop], mm_Wk[hop],
                       mm_Wv[hop], mm_Wskip[hop], mm_bskip[hop])
    h_mov = _res_block(h_mov, mov_resW, mov_resb)
    h_ph = _tconv(h_mov, x_phase, ea_mp, ei_mp,
                  mp_Wq, mp_Wk, mp_Wv, mp_Wskip, mp_bskip)
    h_ph = _tconv(h_ph, h_ph, ea_pp, ei_pp,
                  pp_Wq, pp_Wk, pp_Wv, pp_Wskip, pp_bskip)
    h_ph = _res_block(h_ph, ph_resW, ph_resb)

    src_mi, dst_mi = ei_mov_int[0], ei_mov_int[1]
    mv = h_mov[src_mi] @ W_sv + b_sv
    sv = jax.ops.segment_sum(mv, dst_mi, num_segments=N_INT)
    action_index = ei_phase_int[1]
    sv_a = sv[action_index][:, 0]
    adv = (h_ph @ W_sv + b_sv)[:, 0]
    cnt = jax.ops.segment_sum(jnp.ones_like(adv), action_index,
                              num_segments=N_INT)
    mean_adv = jax.ops.segment_sum(adv, action_index,
                                   num_segments=N_INT) / jnp.maximum(cnt, 1.0)
    adv = adv - mean_adv[action_index]
    return sv_a + adv, action_index


# speedup vs baseline: 22.1794x; 1.8472x over previous
import functools
import math

import jax
import jax.numpy as jnp
from jax import lax
from jax.experimental import pallas as pl
from jax.experimental.pallas import tpu as pltpu
from jax.experimental.pallas import tpu_sc as plsc

HID = 64
HEADS = 8
DH = 8
N_INT = 10000
NC, NS = 2, 16
NW = NC * NS
LB = 128
EGRAN = NW * LB


def _rup(x, m):
    return (x + m - 1) // m * m


def _mesh():
    return plsc.VectorSubcoreMesh(core_axis_name="c", subcore_axis_name="s")


def _mmb(x, W, b):
    n, d = x.shape
    m = W.shape[1]
    BN = 2048
    b2 = b.reshape(1, m)

    def body(x_ref, w_ref, b_ref, o_ref):
        o_ref[...] = jnp.dot(x_ref[...], w_ref[...],
                             preferred_element_type=jnp.float32) + b_ref[...]

    return pl.pallas_call(
        body,
        grid=(pl.cdiv(n, BN),),
        in_specs=[
            pl.BlockSpec((BN, d), lambda i: (i, 0)),
            pl.BlockSpec((d, m), lambda i: (0, 0)),
            pl.BlockSpec((1, m), lambda i: (0, 0)),
        ],
        out_specs=pl.BlockSpec((BN, m), lambda i: (i, 0)),
        out_shape=jax.ShapeDtypeStruct((n, m), jnp.float32),
    )(x, W, b2)


def _res_block(h, Ws, bs):
    nres = Ws.shape[0]

    def body(h_ref, W_ref, b_ref, o_ref):
        hh = h_ref[...]
        for i in range(nres):
            hh = hh + jnp.maximum(
                jnp.dot(hh, W_ref[i], preferred_element_type=jnp.float32)
                + b_ref[i], 0.0)
        o_ref[...] = hh

    n = h.shape[0]
    BN = 2048
    return pl.pallas_call(
        body,
        grid=(pl.cdiv(n, BN),),
        in_specs=[
            pl.BlockSpec((BN, HID), lambda i: (i, 0)),
            pl.BlockSpec((nres, HID, HID), lambda i: (0, 0, 0)),
            pl.BlockSpec((nres, HID), lambda i: (0, 0)),
        ],
        out_specs=pl.BlockSpec((BN, HID), lambda i: (i, 0)),
        out_shape=jax.ShapeDtypeStruct((n, HID), jnp.float32),
    )(h, Ws, bs)


def _edge(kvg, qg, ea, Wk2, Wv2, n_edges):
    Ep = kvg.shape[0]
    de = ea.shape[1]
    BE = 4096
    W128 = 2 * HID

    def body(kv_ref, q_ref, ea_ref, wk_ref, wv_ref, o_ref):
        ii = lax.broadcasted_iota(jnp.int32, (HID, HEADS), 0) // DH
        jj = lax.broadcasted_iota(jnp.int32, (HID, HEADS), 1)
        M = (ii == jj).astype(jnp.float32)
        ai = lax.broadcasted_iota(jnp.int32, (HID, W128), 0)
        aj = lax.broadcasted_iota(jnp.int32, (HID, W128), 1)
        A = (ai == aj).astype(jnp.float32)
        bi = lax.broadcasted_iota(jnp.int32, (HEADS, W128), 0)
        bj = lax.broadcasted_iota(jnp.int32, (HEADS, W128), 1)
        B = (bj == bi + HID).astype(jnp.float32)
        kv = kv_ref[...]
        q = q_ref[...][:, :HID]
        eat = ea_ref[...]
        k = kv[:, :HID] + jnp.dot(eat, wk_ref[...],
                                  preferred_element_type=jnp.float32)
        v = kv[:, HID:] + jnp.dot(eat, wv_ref[...],
                                  preferred_element_type=jnp.float32)
        z = jnp.dot(q * k, M, preferred_element_type=jnp.float32)
        w = jnp.exp(z)
        ridx = pl.program_id(0) * BE + lax.broadcasted_iota(
            jnp.int32, (BE, 1), 0)
        msk = (ridx < n_edges).astype(jnp.float32)
        w = w * msk
        wb = jnp.dot(w, M.T, preferred_element_type=jnp.float32)
        o_ref[...] = (jnp.dot(wb * v, A, preferred_element_type=jnp.float32)
                      + jnp.dot(w, B, preferred_element_type=jnp.float32))

    return pl.pallas_call(
        body,
        grid=(Ep // BE,),
        in_specs=[
            pl.BlockSpec((BE, W128), lambda i: (i, 0)),
            pl.BlockSpec((BE, W128), lambda i: (i, 0)),
            pl.BlockSpec((BE, de), lambda i: (i, 0)),
            pl.BlockSpec((de, HID), lambda i: (0, 0)),
            pl.BlockSpec((de, HID), lambda i: (0, 0)),
        ],
        out_specs=pl.BlockSpec((BE, W128), lambda i: (i, 0)),
        out_shape=jax.ShapeDtypeStruct((Ep, W128), jnp.float32),
    )(kvg, qg, ea, Wk2, Wv2)


def _finalize(acc, S):
    n = S.shape[0]
    BN = 2048
    W128 = 2 * HID

    def body(a_ref, s_ref, o_ref):
        ni = lax.broadcasted_iota(jnp.int32, (W128, HID), 0)
        nj = lax.broadcasted_iota(jnp.int32, (W128, HID), 1)
        NP = (ni == nj).astype(jnp.float32)
        DP = (ni == HID + nj // DH).astype(jnp.float32)
        a = a_ref[0] + a_ref[1]
        num = jnp.dot(a, NP, preferred_element_type=jnp.float32)
        rep = jnp.dot(a, DP, preferred_element_type=jnp.float32)
        o_ref[...] = num / jnp.maximum(rep, 1e-30) + s_ref[...]

    return pl.pallas_call(
        body,
        grid=(pl.cdiv(n, BN),),
        in_specs=[
            pl.BlockSpec((2, BN, W128), lambda i: (0, i, 0)),
            pl.BlockSpec((BN, HID), lambda i: (i, 0)),
        ],
        out_specs=pl.BlockSpec((BN, HID), lambda i: (i, 0)),
        out_shape=jax.ShapeDtypeStruct((n, HID), jnp.float32),
    )(acc, S)


def _combine(accS, accA):
    BN = 2048

    def body(as_ref, aa_ref, o_ref):
        c0 = (lax.broadcasted_iota(jnp.int32, (2 * HID, 1), 0) == 0
              ).astype(jnp.float32)
        c1 = (lax.broadcasted_iota(jnp.int32, (2 * HID, 1), 0) == 1
              ).astype(jnp.float32)
        e0 = (lax.broadcasted_iota(jnp.int32, (1, 2 * HID), 1) == 0
              ).astype(jnp.float32)
        e1 = (lax.broadcasted_iota(jnp.int32, (1, 2 * HID), 1) == 1
              ).astype(jnp.float32)
        s = as_ref[0] + as_ref[1]
        a = aa_ref[0] + aa_ref[1]
        sv = jnp.dot(s, c0, preferred_element_type=jnp.float32)
        mean = (jnp.dot(a, c0, preferred_element_type=jnp.float32)
                / jnp.maximum(jnp.dot(a, c1,
                                      preferred_element_type=jnp.float32),
                              1.0))
        o_ref[...] = sv * e0 + mean * e1

    return pl.pallas_call(
        body,
        grid=(pl.cdiv(N_INT, BN),),
        in_specs=[
            pl.BlockSpec((2, BN, 2 * HID), lambda i: (0, i, 0)),
            pl.BlockSpec((2, BN, 2 * HID), lambda i: (0, i, 0)),
        ],
        out_specs=pl.BlockSpec((BN, 2 * HID), lambda i: (i, 0)),
        out_shape=jax.ShapeDtypeStruct((N_INT, 2 * HID), jnp.float32),
    )(accS, accA)


def _final(G, A, n_out):
    BN = 2048

    def body(g_ref, a_ref, o_ref):
        gc0 = (lax.broadcasted_iota(jnp.int32, (2 * HID, 1), 0) == 0
               ).astype(jnp.float32)
        gc1 = (lax.broadcasted_iota(jnp.int32, (2 * HID, 1), 0) == 1
               ).astype(jnp.float32)
        ac0 = (lax.broadcasted_iota(jnp.int32, (2 * HID, 1), 0) == 0
               ).astype(jnp.float32)
        e0 = (lax.broadcasted_iota(jnp.int32, (1, 16), 1) == 0
              ).astype(jnp.float32)
        g = g_ref[...]
        r = (jnp.dot(g, gc0, preferred_element_type=jnp.float32)
             + jnp.dot(a_ref[...], ac0, preferred_element_type=jnp.float32)
             - jnp.dot(g, gc1, preferred_element_type=jnp.float32))
        o_ref[...] = r * e0

    return pl.pallas_call(
        body,
        grid=(pl.cdiv(n_out, BN),),
        in_specs=[
            pl.BlockSpec((BN, 2 * HID), lambda i: (i, 0)),
            pl.BlockSpec((BN, 2 * HID), lambda i: (i, 0)),
        ],
        out_specs=pl.BlockSpec((BN, 16), lambda i: (i, 0)),
        out_shape=jax.ShapeDtypeStruct((n_out, 16), jnp.float32),
    )(G, A)


def _sc_gather_kvq(KV, Q, src2, dst2):
    rows2 = src2.shape[0]
    per_w = rows2 // NW

    @functools.partial(
        pl.kernel,
        mesh=_mesh(),
        out_type=[
            jax.ShapeDtypeStruct((rows2, LB, 2 * HID), jnp.float32),
            jax.ShapeDtypeStruct((rows2, LB, 2 * HID), jnp.float32),
        ],
        scratch_types=[
            pltpu.VMEM((LB,), jnp.int32),
            pltpu.VMEM((LB,), jnp.int32),
            pltpu.VMEM((LB, 2 * HID), jnp.float32),
            pltpu.VMEM((LB, 2 * HID), jnp.float32),
            pltpu.SemaphoreType.DMA,
            pltpu.SemaphoreType.DMA,
        ],
    )
    def k(kv_hbm, q_hbm, s_hbm, d_hbm, kvg_hbm, qg_hbm,
          sidx, didx, kvb, qb, sem1, sem2):
        wid = lax.axis_index("s") * NC + lax.axis_index("c")
        base = wid * per_w

        def body(i, carry):
            w = base + i
            pltpu.sync_copy(s_hbm.at[w], sidx)
            pltpu.sync_copy(d_hbm.at[w], didx)
            cp1 = pltpu.async_copy(kv_hbm.at[sidx], kvb, sem1)
            cp2 = pltpu.async_copy(q_hbm.at[didx], qb, sem2)
            cp1.wait()
            cp2.wait()
            pltpu.sync_copy(kvb, kvg_hbm.at[w])
            pltpu.sync_copy(qb, qg_hbm.at[w])
            return carry

        lax.fori_loop(0, per_w, body, 0)

    return k(KV, Q, src2, dst2)


def _sc_gather1(T, idx2, width):
    rows2 = idx2.shape[0]
    per_w = rows2 // NW

    @functools.partial(
        pl.kernel,
        mesh=_mesh(),
        out_type=jax.ShapeDtypeStruct((rows2, LB, width), jnp.float32),
        scratch_types=[
            pltpu.VMEM((LB,), jnp.int32),
            pltpu.VMEM((LB, width), jnp.float32),
            pltpu.SemaphoreType.DMA,
        ],
    )
    def k(t_hbm, i_hbm, out_hbm, idxb, rowsb, sem):
        wid = lax.axis_index("s") * NC + lax.axis_index("c")
        base = wid * per_w

        def body(i, carry):
            w = base + i
            pltpu.sync_copy(i_hbm.at[w], idxb)
            pltpu.async_copy(t_hbm.at[idxb], rowsb, sem).wait()
            pltpu.sync_copy(rowsb, out_hbm.at[w])
            return carry

        lax.fori_loop(0, per_w, body, 0)

    return k(T, idx2)


NSUB = 13952
NACC = NSUB + 128
ZROWS = NACC // NS


def _m8(x):
    return pl.multiple_of(x, 8)


def _sc_scatter(wv3, dst2, n_dst, zeros_z):
    W128 = 2 * HID
    rows2 = dst2.shape[0]
    rows_t = rows2 // NW
    ndp = _rup(n_dst, LB)
    npass = -(-ndp // NSUB)

    @functools.partial(
        pl.kernel,
        mesh=_mesh(),
        out_type=jax.ShapeDtypeStruct((NC, ndp, W128), jnp.float32),
        scratch_types=[
            pltpu.VMEM((LB,), jnp.int32),
            pltpu.VMEM((LB,), jnp.int32),
            pltpu.VMEM((LB, W128), jnp.float32),
            pltpu.VMEM_SHARED((NACC, W128), jnp.float32),
        ],
    )
    def k(wv_hbm, d_hbm, z_hbm, out_hbm, idxb, idx2b, rowsb, acc):
        c = lax.axis_index("c")
        t = lax.axis_index("s")
        sbase = (c * NS + t) * rows_t
        trash = NSUB + (lax.iota(jnp.int32, 16) & 15)

        for p in range(npass):
            base_row = p * NSUB
            cnt = min(ndp - base_row, NSUB)
            wcnt = cnt // NS
            pltpu.sync_copy(z_hbm, acc.at[pl.ds(_m8(t * ZROWS), ZROWS)])
            plsc.subcore_barrier()

            def sb(i, carry):
                w = sbase + i
                pltpu.sync_copy(d_hbm.at[w], idxb)
                pltpu.sync_copy(wv_hbm.at[w], rowsb)
                for g in range(LB // 16):
                    tv = idxb[pl.ds(g * 16, 16)] - base_row
                    m = (tv >= 0) & (tv < cnt)
                    idx2b[pl.ds(g * 16, 16)] = jnp.where(m, tv, trash)
                pltpu.sync_copy(rowsb, acc.at[idx2b], add=True)
                return carry
            lax.fori_loop(0, rows_t, sb, 0)
            plsc.subcore_barrier()

            wr0 = t * wcnt
            nfull = wcnt // LB
            rem = wcnt % LB

            def wb(i, carry):
                pltpu.sync_copy(
                    acc.at[pl.ds(_m8(wr0 + i * LB), LB)],
                    out_hbm.at[c, pl.ds(_m8(base_row + wr0 + i * LB), LB),
                               :])
                return carry
            lax.fori_loop(0, nfull, wb, 0)
            if rem:
                pltpu.sync_copy(
                    acc.at[pl.ds(_m8(wr0 + nfull * LB), rem)],
                    out_hbm.at[c,
                               pl.ds(_m8(base_row + wr0 + nfull * LB), rem),
                               :])
            plsc.subcore_barrier()

    return k(wv3, dst2, zeros_z)


def _pad_idx(idx, n_rows, Ep):
    E = idx.shape[0]
    pad = Ep - E
    idx = idx.astype(jnp.int32)
    if pad:
        fill = jnp.arange(pad, dtype=jnp.int32) % n_rows
        idx = jnp.concatenate([idx, fill])
    return idx.reshape(Ep // LB, LB)


def _conv(x_src, x_dst, ea, ei, Wq, Wk, Wv, Wskip, bskip, zz):
    n_src, d_src = x_src.shape
    n_dst = x_dst.shape[0]
    E = ei.shape[1]
    Ep = _rup(E, EGRAN)
    rows2 = Ep // LB
    src2 = _pad_idx(ei[0], n_src, Ep)
    dst2 = _pad_idx(ei[1], n_dst, Ep)

    WKV = jnp.concatenate([Wk[:d_src], Wv[:d_src]], axis=1)
    zero128 = jnp.zeros((2 * HID,), jnp.float32)
    KV = _mmb(x_src, WKV, zero128)
    scale = 1.0 / math.sqrt(float(DH))
    WQ128 = jnp.concatenate(
        [Wq * scale, jnp.zeros((Wq.shape[0], HID), jnp.float32)], axis=1)
    Q = _mmb(x_dst, WQ128, zero128)
    S = _mmb(x_dst, Wskip, bskip)

    KVg, Qg = _sc_gather_kvq(KV, Q, src2, dst2)
    wv = _edge(KVg.reshape(Ep, 2 * HID), Qg.reshape(Ep, 2 * HID),
               ea, Wk[d_src:], Wv[d_src:], E)
    acc = _sc_scatter(wv.reshape(rows2, LB, 2 * HID), dst2, n_dst, zz)
    return _finalize(acc, S)


def kernel(x_lane_segment, x_lane, x_movement, x_phase,
           ea_ls_lane, ea_lane_down, ea_lane_up, ea_mm, ea_mp, ea_pp,
           ei_ls_lane, ei_lane_down, ei_lane_up, ei_mm, ei_mp, ei_pp,
           ei_mov_int, ei_phase_int,
           lane_Wq, lane_Wk, lane_Wv, lane_Wskip, lane_bskip,
           mvd_Wq, mvd_Wk, mvd_Wv, mvd_Wskip, mvd_bskip,
           mvu_Wq, mvu_Wk, mvu_Wv, mvu_Wskip, mvu_bskip,
           mm_Wq, mm_Wk, mm_Wv, mm_Wskip, mm_bskip,
           mp_Wq, mp_Wk, mp_Wv, mp_Wskip, mp_bskip,
           pp_Wq, pp_Wk, pp_Wv, pp_Wskip, pp_bskip,
           lane_resW, lane_resb, mov_resW, mov_resb, ph_resW, ph_resb,
           W_sv, b_sv):
    zz = jnp.zeros((ZROWS, 2 * HID), jnp.float32)
    h_lane = _conv(x_lane_segment, x_lane, ea_ls_lane, ei_ls_lane,
                   lane_Wq, lane_Wk, lane_Wv, lane_Wskip, lane_bskip, zz)
    h_lane = _res_block(h_lane, lane_resW, lane_resb)
    h_mov = _conv(h_lane, x_movement, ea_lane_down, ei_lane_down,
                  mvd_Wq, mvd_Wk, mvd_Wv, mvd_Wskip, mvd_bskip, zz)
    h_mov = h_mov + _conv(h_lane, x_movement, ea_lane_up, ei_lane_up,
                          mvu_Wq, mvu_Wk, mvu_Wv, mvu_Wskip, mvu_bskip, zz)
    for hop in range(2):
        h_mov = _conv(h_mov, h_mov, ea_mm, ei_mm, mm_Wq[hop], mm_Wk[hop],
                      mm_Wv[hop], mm_Wskip[hop], mm_bskip[hop], zz)
    h_mov = _res_block(h_mov, mov_resW, mov_resb)
    h_ph = _conv(h_mov, x_phase, ea_mp, ei_mp,
                 mp_Wq, mp_Wk, mp_Wv, mp_Wskip, mp_bskip, zz)
    h_ph = _conv(h_ph, h_ph, ea_pp, ei_pp,
                 pp_Wq, pp_Wk, pp_Wv, pp_Wskip, pp_bskip, zz)
    h_ph = _res_block(h_ph, ph_resW, ph_resb)

    n_mov = x_movement.shape[0]
    W128 = 2 * HID
    Wsv128 = jnp.concatenate(
        [W_sv, jnp.zeros((HID, W128 - 1), jnp.float32)], axis=1)
    b128 = jnp.concatenate([b_sv, jnp.zeros((W128 - 1,), jnp.float32)])
    mvn = _mmb(h_mov, Wsv128, b128)
    mvnp = jnp.concatenate(
        [mvn, jnp.zeros((16, W128), jnp.float32)], axis=0)

    E_mi = ei_mov_int.shape[1]
    Ep_mi = _rup(E_mi, EGRAN)
    pad_mi = Ep_mi - E_mi
    src_mi = jnp.concatenate([
        ei_mov_int[0].astype(jnp.int32),
        n_mov + (jnp.arange(pad_mi, dtype=jnp.int32) % 16),
    ]).reshape(Ep_mi // LB, LB)
    dst_mi = _pad_idx(ei_mov_int[1], N_INT, Ep_mi)
    Hg = _sc_gather1(mvnp, src_mi, W128)
    accS = _sc_scatter(Hg, dst_mi, N_INT, zz)

    b128a = jnp.concatenate([b_sv, jnp.ones((1,), jnp.float32),
                             jnp.zeros((W128 - 2,), jnp.float32)])
    advn = _mmb(h_ph, Wsv128, b128a)
    E_pi = ei_phase_int.shape[1]
    Ep_pi = _rup(E_pi, EGRAN)
    act2 = _pad_idx(ei_phase_int[1], N_INT, Ep_pi)
    advp = jnp.concatenate(
        [advn, jnp.zeros((Ep_pi - E_pi, W128), jnp.float32)], axis=0)
    accA = _sc_scatter(advp.reshape(Ep_pi // LB, LB, W128), act2, N_INT, zz)

    T16 = _combine(accS, accA)
    G = _sc_gather1(T16, act2, W128)
    res = _final(G.reshape(Ep_pi, W128)[:E_pi], advn, E_pi)
    return res[:, 0], ei_phase_int[1]


# speedup vs baseline: 28.8627x; 1.3013x over previous
import functools
import math

import jax
import jax.numpy as jnp
from jax import lax
from jax.experimental import pallas as pl
from jax.experimental.pallas import tpu as pltpu
from jax.experimental.pallas import tpu_sc as plsc

HID = 64
HEADS = 8
DH = 8
N_INT = 10000
NC, NS = 2, 16
NW = NC * NS
LB = 128
EGRAN = NW * LB


def _rup(x, m):
    return (x + m - 1) // m * m


def _mesh():
    return plsc.VectorSubcoreMesh(core_axis_name="c", subcore_axis_name="s")


def _mmb(x, W, b):
    n, d = x.shape
    m = W.shape[1]
    BN = 2048
    b2 = b.reshape(1, m)

    def body(x_ref, w_ref, b_ref, o_ref):
        o_ref[...] = jnp.dot(x_ref[...], w_ref[...],
                             preferred_element_type=jnp.float32) + b_ref[...]

    return pl.pallas_call(
        body,
        grid=(pl.cdiv(n, BN),),
        in_specs=[
            pl.BlockSpec((BN, d), lambda i: (i, 0)),
            pl.BlockSpec((d, m), lambda i: (0, 0)),
            pl.BlockSpec((1, m), lambda i: (0, 0)),
        ],
        out_specs=pl.BlockSpec((BN, m), lambda i: (i, 0)),
        out_shape=jax.ShapeDtypeStruct((n, m), jnp.float32),
    )(x, W, b2)


def _res_block(h, Ws, bs):
    nres = Ws.shape[0]

    def body(h_ref, W_ref, b_ref, o_ref):
        hh = h_ref[...]
        for i in range(nres):
            hh = hh + jnp.maximum(
                jnp.dot(hh, W_ref[i], preferred_element_type=jnp.float32)
                + b_ref[i], 0.0)
        o_ref[...] = hh

    n = h.shape[0]
    BN = 2048
    return pl.pallas_call(
        body,
        grid=(pl.cdiv(n, BN),),
        in_specs=[
            pl.BlockSpec((BN, HID), lambda i: (i, 0)),
            pl.BlockSpec((nres, HID, HID), lambda i: (0, 0, 0)),
            pl.BlockSpec((nres, HID), lambda i: (0, 0)),
        ],
        out_specs=pl.BlockSpec((BN, HID), lambda i: (i, 0)),
        out_shape=jax.ShapeDtypeStruct((n, HID), jnp.float32),
    )(h, Ws, bs)


def _edge(kvg, qg, ea, Wk2, Wv2, n_edges):
    Ep = kvg.shape[0]
    de = ea.shape[1]
    BE = 4096
    W128 = 2 * HID

    def body(kv_ref, q_ref, ea_ref, wk_ref, wv_ref, o_ref):
        ii = lax.broadcasted_iota(jnp.int32, (HID, HEADS), 0) // DH
        jj = lax.broadcasted_iota(jnp.int32, (HID, HEADS), 1)
        M = (ii == jj).astype(jnp.float32)
        ai = lax.broadcasted_iota(jnp.int32, (HID, W128), 0)
        aj = lax.broadcasted_iota(jnp.int32, (HID, W128), 1)
        A = (ai == aj).astype(jnp.float32)
        bi = lax.broadcasted_iota(jnp.int32, (HEADS, W128), 0)
        bj = lax.broadcasted_iota(jnp.int32, (HEADS, W128), 1)
        B = (bj == bi + HID).astype(jnp.float32)
        kv = kv_ref[...]
        q = q_ref[...][:, :HID]
        eat = ea_ref[...]
        k = kv[:, :HID] + jnp.dot(eat, wk_ref[...],
                                  preferred_element_type=jnp.float32)
        v = kv[:, HID:] + jnp.dot(eat, wv_ref[...],
                                  preferred_element_type=jnp.float32)
        z = jnp.dot(q * k, M, preferred_element_type=jnp.float32)
        w = jnp.exp(z)
        ridx = pl.program_id(0) * BE + lax.broadcasted_iota(
            jnp.int32, (BE, 1), 0)
        msk = (ridx < n_edges).astype(jnp.float32)
        w = w * msk
        wb = jnp.dot(w, M.T, preferred_element_type=jnp.float32)
        o_ref[...] = (jnp.dot(wb * v, A, preferred_element_type=jnp.float32)
                      + jnp.dot(w, B, preferred_element_type=jnp.float32))

    return pl.pallas_call(
        body,
        grid=(Ep // BE,),
        in_specs=[
            pl.BlockSpec((BE, W128), lambda i: (i, 0)),
            pl.BlockSpec((BE, W128), lambda i: (i, 0)),
            pl.BlockSpec((BE, de), lambda i: (i, 0)),
            pl.BlockSpec((de, HID), lambda i: (0, 0)),
            pl.BlockSpec((de, HID), lambda i: (0, 0)),
        ],
        out_specs=pl.BlockSpec((BE, W128), lambda i: (i, 0)),
        out_shape=jax.ShapeDtypeStruct((Ep, W128), jnp.float32),
    )(kvg, qg, ea, Wk2, Wv2)


def _finalize(acc, S):
    n = S.shape[0]
    BN = 2048
    W128 = 2 * HID

    def body(a_ref, s_ref, o_ref):
        ni = lax.broadcasted_iota(jnp.int32, (W128, HID), 0)
        nj = lax.broadcasted_iota(jnp.int32, (W128, HID), 1)
        NP = (ni == nj).astype(jnp.float32)
        DP = (ni == HID + nj // DH).astype(jnp.float32)
        a = a_ref[0] + a_ref[1]
        num = jnp.dot(a, NP, preferred_element_type=jnp.float32)
        rep = jnp.dot(a, DP, preferred_element_type=jnp.float32)
        o_ref[...] = num / jnp.maximum(rep, 1e-30) + s_ref[...]

    return pl.pallas_call(
        body,
        grid=(pl.cdiv(n, BN),),
        in_specs=[
            pl.BlockSpec((2, BN, W128), lambda i: (0, i, 0)),
            pl.BlockSpec((BN, HID), lambda i: (i, 0)),
        ],
        out_specs=pl.BlockSpec((BN, HID), lambda i: (i, 0)),
        out_shape=jax.ShapeDtypeStruct((n, HID), jnp.float32),
    )(acc, S)


def _combine(accS, accA):
    BN = 2048

    def body(as_ref, aa_ref, o_ref):
        c0 = (lax.broadcasted_iota(jnp.int32, (2 * HID, 1), 0) == 0
              ).astype(jnp.float32)
        c1 = (lax.broadcasted_iota(jnp.int32, (2 * HID, 1), 0) == 1
              ).astype(jnp.float32)
        e0 = (lax.broadcasted_iota(jnp.int32, (1, 2 * HID), 1) == 0
              ).astype(jnp.float32)
        e1 = (lax.broadcasted_iota(jnp.int32, (1, 2 * HID), 1) == 1
              ).astype(jnp.float32)
        s = as_ref[0] + as_ref[1]
        a = aa_ref[0] + aa_ref[1]
        sv = jnp.dot(s, c0, preferred_element_type=jnp.float32)
        mean = (jnp.dot(a, c0, preferred_element_type=jnp.float32)
                / jnp.maximum(jnp.dot(a, c1,
                                      preferred_element_type=jnp.float32),
                              1.0))
        o_ref[...] = sv * e0 + mean * e1

    return pl.pallas_call(
        body,
        grid=(pl.cdiv(N_INT, BN),),
        in_specs=[
            pl.BlockSpec((2, BN, 2 * HID), lambda i: (0, i, 0)),
            pl.BlockSpec((2, BN, 2 * HID), lambda i: (0, i, 0)),
        ],
        out_specs=pl.BlockSpec((BN, 2 * HID), lambda i: (i, 0)),
        out_shape=jax.ShapeDtypeStruct((N_INT, 2 * HID), jnp.float32),
    )(accS, accA)


def _final(G, A, n_out):
    BN = 2048

    def body(g_ref, a_ref, o_ref):
        gc0 = (lax.broadcasted_iota(jnp.int32, (2 * HID, 1), 0) == 0
               ).astype(jnp.float32)
        gc1 = (lax.broadcasted_iota(jnp.int32, (2 * HID, 1), 0) == 1
               ).astype(jnp.float32)
        ac0 = (lax.broadcasted_iota(jnp.int32, (2 * HID, 1), 0) == 0
               ).astype(jnp.float32)
        e0 = (lax.broadcasted_iota(jnp.int32, (1, 16), 1) == 0
              ).astype(jnp.float32)
        g = g_ref[...]
        r = (jnp.dot(g, gc0, preferred_element_type=jnp.float32)
             + jnp.dot(a_ref[...], ac0, preferred_element_type=jnp.float32)
             - jnp.dot(g, gc1, preferred_element_type=jnp.float32))
        o_ref[...] = r * e0

    return pl.pallas_call(
        body,
        grid=(pl.cdiv(n_out, BN),),
        in_specs=[
            pl.BlockSpec((BN, 2 * HID), lambda i: (i, 0)),
            pl.BlockSpec((BN, 2 * HID), lambda i: (i, 0)),
        ],
        out_specs=pl.BlockSpec((BN, 16), lambda i: (i, 0)),
        out_shape=jax.ShapeDtypeStruct((n_out, 16), jnp.float32),
    )(G, A)


def _sc_gather_kvq(KV, Q, src2, dst2):
    rows2 = src2.shape[0]
    per_w = rows2 // NW

    @functools.partial(
        pl.kernel,
        mesh=_mesh(),
        out_type=[
            jax.ShapeDtypeStruct((rows2, LB, 2 * HID), jnp.float32),
            jax.ShapeDtypeStruct((rows2, LB, 2 * HID), jnp.float32),
        ],
        scratch_types=[
            pltpu.VMEM((2, LB), jnp.int32),
            pltpu.VMEM((2, LB), jnp.int32),
            pltpu.VMEM((2, LB, 2 * HID), jnp.float32),
            pltpu.VMEM((2, LB, 2 * HID), jnp.float32),
            pltpu.SemaphoreType.DMA,
            pltpu.SemaphoreType.DMA,
            pltpu.SemaphoreType.DMA,
            pltpu.SemaphoreType.DMA,
        ],
    )
    def k(kv_hbm, q_hbm, s_hbm, d_hbm, kvg_hbm, qg_hbm,
          sidx, didx, kvb, qb, semi0, semi1, semg0, semg1):
        wid = lax.axis_index("s") * NC + lax.axis_index("c")
        base = wid * per_w
        semi = (semi0, semi1)
        semg = (semg0, semg1)

        def idx_load(w, b):
            pltpu.async_copy(s_hbm.at[w], sidx.at[b], semi[b])
            pltpu.async_copy(d_hbm.at[w], didx.at[b], semi[b])

        def idx_wait(w, b):
            pltpu.make_async_copy(s_hbm.at[w], sidx.at[b], semi[b]).wait()
            pltpu.make_async_copy(d_hbm.at[w], didx.at[b], semi[b]).wait()

        def gather_start(b):
            pltpu.async_copy(kv_hbm.at[sidx.at[b]], kvb.at[b], semg[b])
            pltpu.async_copy(q_hbm.at[didx.at[b]], qb.at[b], semg[b])

        def gather_wait(b):
            pltpu.make_async_copy(kv_hbm.at[sidx.at[b]], kvb.at[b],
                                  semg[b]).wait()
            pltpu.make_async_copy(q_hbm.at[didx.at[b]], qb.at[b],
                                  semg[b]).wait()

        def writeback(w, b):
            pltpu.sync_copy(kvb.at[b], kvg_hbm.at[w])
            pltpu.sync_copy(qb.at[b], qg_hbm.at[w])

        idx_load(base, 0)

        def pair(j, carry):
            for b in (0, 1):
                w = base + 2 * j + b
                idx_wait(w, b)
                gather_start(b)
                if b == 1:
                    gather_wait(0)
                    writeback(w - 1, 0)
                else:
                    @pl.when(j > 0)
                    def _():
                        gather_wait(1)
                        writeback(w - 1, 1)

                @pl.when(2 * j + b + 1 < per_w)
                def _():
                    idx_load(w + 1, 1 - b)
            return carry

        lax.fori_loop(0, per_w // 2, pair, 0)
        last = per_w - 1
        lb = last & 1
        if per_w % 2:
            w = base + last
            idx_wait(w, lb)
            gather_start(lb)
            gather_wait(1 - lb)
            writeback(w - 1, 1 - lb)
        gather_wait(lb)
        writeback(base + last, lb)

    return k(KV, Q, src2, dst2)


def _sc_gather1(T, idx2, width):
    rows2 = idx2.shape[0]
    per_w = rows2 // NW

    @functools.partial(
        pl.kernel,
        mesh=_mesh(),
        out_type=jax.ShapeDtypeStruct((rows2, LB, width), jnp.float32),
        scratch_types=[
            pltpu.VMEM((2, LB), jnp.int32),
            pltpu.VMEM((2, LB, width), jnp.float32),
            pltpu.SemaphoreType.DMA,
            pltpu.SemaphoreType.DMA,
            pltpu.SemaphoreType.DMA,
            pltpu.SemaphoreType.DMA,
        ],
    )
    def k(t_hbm, i_hbm, out_hbm, idxb, rowsb, semi0, semi1, semg0, semg1):
        wid = lax.axis_index("s") * NC + lax.axis_index("c")
        base = wid * per_w
        semi = (semi0, semi1)
        semg = (semg0, semg1)

        def idx_load(w, b):
            pltpu.async_copy(i_hbm.at[w], idxb.at[b], semi[b])

        def idx_wait(w, b):
            pltpu.make_async_copy(i_hbm.at[w], idxb.at[b], semi[b]).wait()

        def gather_start(b):
            pltpu.async_copy(t_hbm.at[idxb.at[b]], rowsb.at[b], semg[b])

        def gather_wait(b):
            pltpu.make_async_copy(t_hbm.at[idxb.at[b]], rowsb.at[b],
                                  semg[b]).wait()

        idx_load(base, 0)

        def pair(j, carry):
            for b in (0, 1):
                w = base + 2 * j + b
                idx_wait(w, b)
                gather_start(b)
                if b == 1:
                    gather_wait(0)
                    pltpu.sync_copy(rowsb.at[0], out_hbm.at[w - 1])
                else:
                    @pl.when(j > 0)
                    def _():
                        gather_wait(1)
                        pltpu.sync_copy(rowsb.at[1], out_hbm.at[w - 1])

                @pl.when(2 * j + b + 1 < per_w)
                def _():
                    idx_load(w + 1, 1 - b)
            return carry

        lax.fori_loop(0, per_w // 2, pair, 0)
        last = per_w - 1
        lb = last & 1
        if per_w % 2:
            w = base + last
            idx_wait(w, lb)
            gather_start(lb)
            gather_wait(1 - lb)
            pltpu.sync_copy(rowsb.at[1 - lb], out_hbm.at[w - 1])
        gather_wait(lb)
        pltpu.sync_copy(rowsb.at[lb], out_hbm.at[base + last])

    return k(T, idx2)


NSUB = 12032
NACC = NSUB + 128
ZROWS = NACC // NS


def _m8(x):
    return pl.multiple_of(x, 8)


def _sc_scatter(wv3, dst2, n_dst, zeros_z):
    W128 = 2 * HID
    rows2 = dst2.shape[0]
    rows_t = rows2 // NW
    ndp = _rup(n_dst, LB)
    npass = -(-ndp // NSUB)

    @functools.partial(
        pl.kernel,
        mesh=_mesh(),
        out_type=jax.ShapeDtypeStruct((NC, ndp, W128), jnp.float32),
        scratch_types=[
            pltpu.VMEM((2, LB), jnp.int32),
            pltpu.VMEM((LB,), jnp.int32),
            pltpu.VMEM((2, LB, W128), jnp.float32),
            pltpu.VMEM_SHARED((NACC, W128), jnp.float32),
            pltpu.SemaphoreType.DMA,
            pltpu.SemaphoreType.DMA,
        ],
    )
    def k(wv_hbm, d_hbm, z_hbm, out_hbm, idxb, idx2b, rowsb, acc,
          seml0, seml1):
        c = lax.axis_index("c")
        t = lax.axis_index("s")
        sbase = (c * NS + t) * rows_t
        trash = NSUB + (lax.iota(jnp.int32, 16) & 15)
        seml = (seml0, seml1)

        def load(w, b):
            pltpu.async_copy(d_hbm.at[w], idxb.at[b], seml[b])
            pltpu.async_copy(wv_hbm.at[w], rowsb.at[b], seml[b])

        def load_wait(w, b):
            pltpu.make_async_copy(d_hbm.at[w], idxb.at[b], seml[b]).wait()
            pltpu.make_async_copy(wv_hbm.at[w], rowsb.at[b], seml[b]).wait()

        for p in range(npass):
            base_row = p * NSUB
            cnt = min(ndp - base_row, NSUB)
            wcnt = cnt // NS
            pltpu.sync_copy(z_hbm, acc.at[pl.ds(_m8(t * ZROWS), ZROWS)])
            plsc.subcore_barrier()

            def scat(w, b):
                for g in range(LB // 16):
                    tv = idxb[b, pl.ds(g * 16, 16)] - base_row
                    m = (tv >= 0) & (tv < cnt)
                    idx2b[pl.ds(g * 16, 16)] = jnp.where(m, tv, trash)
                pltpu.sync_copy(rowsb.at[b], acc.at[idx2b], add=True)

            load(sbase, 0)

            def pair(j, carry):
                for b in (0, 1):
                    w = sbase + 2 * j + b
                    load_wait(w, b)

                    @pl.when(2 * j + b + 1 < rows_t)
                    def _():
                        load(w + 1, 1 - b)
                    scat(w, b)
                return carry
            lax.fori_loop(0, rows_t // 2, pair, 0)
            if rows_t % 2:
                w = sbase + rows_t - 1
                lb = (rows_t - 1) & 1
                load_wait(w, lb)
                scat(w, lb)
            plsc.subcore_barrier()

            wr0 = t * wcnt
            nfull = wcnt // LB
            rem = wcnt % LB

            def wb(i, carry):
                pltpu.sync_copy(
                    acc.at[pl.ds(_m8(wr0 + i * LB), LB)],
                    out_hbm.at[c, pl.ds(_m8(base_row + wr0 + i * LB), LB),
                               :])
                return carry
            lax.fori_loop(0, nfull, wb, 0)
            if rem:
                pltpu.sync_copy(
                    acc.at[pl.ds(_m8(wr0 + nfull * LB), rem)],
                    out_hbm.at[c,
                               pl.ds(_m8(base_row + wr0 + nfull * LB), rem),
                               :])
            plsc.subcore_barrier()

    return k(wv3, dst2, zeros_z)


def _pad_idx(idx, n_rows, Ep):
    E = idx.shape[0]
    pad = Ep - E
    idx = idx.astype(jnp.int32)
    if pad:
        fill = jnp.arange(pad, dtype=jnp.int32) % n_rows
        idx = jnp.concatenate([idx, fill])
    return idx.reshape(Ep // LB, LB)


def _conv(x_src, x_dst, ea, ei, Wq, Wk, Wv, Wskip, bskip, zz):
    n_src, d_src = x_src.shape
    n_dst = x_dst.shape[0]
    E = ei.shape[1]
    Ep = _rup(E, EGRAN)
    rows2 = Ep // LB
    src2 = _pad_idx(ei[0], n_src, Ep)
    dst2 = _pad_idx(ei[1], n_dst, Ep)

    WKV = jnp.concatenate([Wk[:d_src], Wv[:d_src]], axis=1)
    zero128 = jnp.zeros((2 * HID,), jnp.float32)
    KV = _mmb(x_src, WKV, zero128)
    scale = 1.0 / math.sqrt(float(DH))
    WQ128 = jnp.concatenate(
        [Wq * scale, jnp.zeros((Wq.shape[0], HID), jnp.float32)], axis=1)
    Q = _mmb(x_dst, WQ128, zero128)
    S = _mmb(x_dst, Wskip, bskip)

    KVg, Qg = _sc_gather_kvq(KV, Q, src2, dst2)
    wv = _edge(KVg.reshape(Ep, 2 * HID), Qg.reshape(Ep, 2 * HID),
               ea, Wk[d_src:], Wv[d_src:], E)
    acc = _sc_scatter(wv.reshape(rows2, LB, 2 * HID), dst2, n_dst, zz)
    return _finalize(acc, S)


def kernel(x_lane_segment, x_lane, x_movement, x_phase,
           ea_ls_lane, ea_lane_down, ea_lane_up, ea_mm, ea_mp, ea_pp,
           ei_ls_lane, ei_lane_down, ei_lane_up, ei_mm, ei_mp, ei_pp,
           ei_mov_int, ei_phase_int,
           lane_Wq, lane_Wk, lane_Wv, lane_Wskip, lane_bskip,
           mvd_Wq, mvd_Wk, mvd_Wv, mvd_Wskip, mvd_bskip,
           mvu_Wq, mvu_Wk, mvu_Wv, mvu_Wskip, mvu_bskip,
           mm_Wq, mm_Wk, mm_Wv, mm_Wskip, mm_bskip,
           mp_Wq, mp_Wk, mp_Wv, mp_Wskip, mp_bskip,
           pp_Wq, pp_Wk, pp_Wv, pp_Wskip, pp_bskip,
           lane_resW, lane_resb, mov_resW, mov_resb, ph_resW, ph_resb,
           W_sv, b_sv):
    zz = jnp.zeros((ZROWS, 2 * HID), jnp.float32)
    h_lane = _conv(x_lane_segment, x_lane, ea_ls_lane, ei_ls_lane,
                   lane_Wq, lane_Wk, lane_Wv, lane_Wskip, lane_bskip, zz)
    h_lane = _res_block(h_lane, lane_resW, lane_resb)
    h_mov = _conv(h_lane, x_movement, ea_lane_down, ei_lane_down,
                  mvd_Wq, mvd_Wk, mvd_Wv, mvd_Wskip, mvd_bskip, zz)
    h_mov = h_mov + _conv(h_lane, x_movement, ea_lane_up, ei_lane_up,
                          mvu_Wq, mvu_Wk, mvu_Wv, mvu_Wskip, mvu_bskip, zz)
    for hop in range(2):
        h_mov = _conv(h_mov, h_mov, ea_mm, ei_mm, mm_Wq[hop], mm_Wk[hop],
                      mm_Wv[hop], mm_Wskip[hop], mm_bskip[hop], zz)
    h_mov = _res_block(h_mov, mov_resW, mov_resb)
    h_ph = _conv(h_mov, x_phase, ea_mp, ei_mp,
                 mp_Wq, mp_Wk, mp_Wv, mp_Wskip, mp_bskip, zz)
    h_ph = _conv(h_ph, h_ph, ea_pp, ei_pp,
                 pp_Wq, pp_Wk, pp_Wv, pp_Wskip, pp_bskip, zz)
    h_ph = _res_block(h_ph, ph_resW, ph_resb)

    n_mov = x_movement.shape[0]
    W128 = 2 * HID
    Wsv128 = jnp.concatenate(
        [W_sv, jnp.zeros((HID, W128 - 1), jnp.float32)], axis=1)
    b128 = jnp.concatenate([b_sv, jnp.zeros((W128 - 1,), jnp.float32)])
    mvn = _mmb(h_mov, Wsv128, b128)
    mvnp = jnp.concatenate(
        [mvn, jnp.zeros((16, W128), jnp.float32)], axis=0)

    E_mi = ei_mov_int.shape[1]
    Ep_mi = _rup(E_mi, EGRAN)
    pad_mi = Ep_mi - E_mi
    src_mi = jnp.concatenate([
        ei_mov_int[0].astype(jnp.int32),
        n_mov + (jnp.arange(pad_mi, dtype=jnp.int32) % 16),
    ]).reshape(Ep_mi // LB, LB)
    dst_mi = _pad_idx(ei_mov_int[1], N_INT, Ep_mi)
    Hg = _sc_gather1(mvnp, src_mi, W128)
    accS = _sc_scatter(Hg, dst_mi, N_INT, zz)

    b128a = jnp.concatenate([b_sv, jnp.ones((1,), jnp.float32),
                             jnp.zeros((W128 - 2,), jnp.float32)])
    advn = _mmb(h_ph, Wsv128, b128a)
    E_pi = ei_phase_int.shape[1]
    Ep_pi = _rup(E_pi, EGRAN)
    act2 = _pad_idx(ei_phase_int[1], N_INT, Ep_pi)
    advp = jnp.concatenate(
        [advn, jnp.zeros((Ep_pi - E_pi, W128), jnp.float32)], axis=0)
    accA = _sc_scatter(advp.reshape(Ep_pi // LB, LB, W128), act2, N_INT, zz)

    T16 = _combine(accS, accA)
    G = _sc_gather1(T16, act2, W128)
    res = _final(G.reshape(Ep_pi, W128)[:E_pi], advn, E_pi)
    return res[:, 0], ei_phase_int[1]
